# Initial kernel scaffold; baseline (speedup 1.0000x reference)
import jax
import jax.numpy as jnp
from jax.experimental import pallas as pl


def kernel(x, edge_index, W1, b1, W2, b2, gru1_wih, gru1_whh, gru1_bih, gru1_bhh, wt1_w, wt1_b, gcn1_b, mem1, gru2_wih, gru2_whh, gru2_bih, gru2_bhh, wt2_w, wt2_b, gcn2_b, mem2, Wp, bp):
    raise NotImplementedError("write your pallas kernel here")


# speedup vs baseline: 62.2843x; 62.2843x over previous
import functools

import jax
import jax.numpy as jnp
from jax import lax
from jax.experimental import pallas as pl
from jax.experimental.pallas import tpu as pltpu
from jax.experimental.pallas import tpu_sc as plsc

N = 100000
E = 3200000
D_IN = 128
H = 16

NC = 2
NS = 16
NW = NC * NS

R = 2048
GRID = 49
NP = R * GRID
NSL = NP // NS

EP = 3276800
EPR = EP // 128
RPW = EPR // NW
CHR = 8
TIT = RPW // CHR


def _leaky(v):
    return jnp.where(v >= 0, v, 0.01 * v)


def _dyn_weight(wih3, bih2, bhh2, mem2, wtw3, wtb2):
    m3 = mem2.reshape(1, 1, H)
    gi = jnp.sum(wih3 * m3, axis=-1) + bih2
    r = jax.nn.sigmoid(gi[0:1] + bhh2[0:1])
    z = jax.nn.sigmoid(gi[1:2] + bhh2[1:2])
    n = jnp.tanh(gi[2:3] + r * bhh2[2:3])
    upd = (1.0 - z) * n
    return jnp.sum(wtw3 * upd.reshape(1, 1, H), axis=-1) + wtb2


def _dis_from_deg(degr):
    deg = degr[0, :] + degr[1, :] + 1.0
    return lax.rsqrt(deg)


def _tc_a_body(xr, w1r, b1r, w2r, b2r, degr, wih3, bih2, bhh2, mem2, wtw3,
               wtb2, outr):
    h = _leaky(lax.dot_general(xr[...], w1r[...], (((1,), (1,)), ((), ())),
                               preferred_element_type=jnp.float32) + b1r[...])
    h = _leaky(lax.dot_general(h, w2r[...], (((1,), (1,)), ((), ())),
                               preferred_element_type=jnp.float32) + b2r[...])
    dis = _dis_from_deg(degr[...])
    nw = _dyn_weight(wih3[...], bih2[...], bhh2[...], mem2[...], wtw3[...],
                     wtb2[...])
    xw = lax.dot_general(h, nw, (((1,), (1,)), ((), ())),
                         preferred_element_type=jnp.float32)
    outr[...] = dis[:, None] * xw


def _tc_mid_body(pr, xwr, degr, gbr, wih3, bih2, bhh2, mem2, wtw3, wtb2, outr):
    dis = _dis_from_deg(degr[...])
    agg = dis[:, None] * (pr[0] + pr[1] + xwr[...]) + gbr[...]
    h = _leaky(agg)
    nw = _dyn_weight(wih3[...], bih2[...], bhh2[...], mem2[...], wtw3[...],
                     wtb2[...])
    xw = lax.dot_general(h, nw, (((1,), (1,)), ((), ())),
                         preferred_element_type=jnp.float32)
    outr[...] = dis[:, None] * xw


def _tc_out_body(qr, xwr, degr, gbr, wpr, bpr, outr):
    dis = _dis_from_deg(degr[...])
    h = _leaky(dis[:, None] * (qr[0] + qr[1] + xwr[...]) + gbr[...])
    outr[...] = jnp.sum(h * wpr[...], axis=1) + bpr[0, 0]


def _full(shape):
    return pl.BlockSpec(shape, lambda i: tuple(0 for _ in shape))


def _tc_a(x, W1, b1, W2, b2, degp, g1):
    return pl.pallas_call(
        _tc_a_body,
        grid=(GRID,),
        in_specs=[
            pl.BlockSpec((R, D_IN), lambda i: (i, 0)),
            _full((256, D_IN)), _full((1, 256)),
            _full((H, 256)), _full((1, H)),
            pl.BlockSpec((NC, R), lambda i: (0, i)),
            _full((3, H, H)), _full((3, H)), _full((3, H)), _full((1, H)),
            _full((H, H, H)), _full((H, H)),
        ],
        out_specs=pl.BlockSpec((R, H), lambda i: (i, 0)),
        out_shape=jax.ShapeDtypeStruct((NP, H), jnp.float32),
    )(x, W1, b1.reshape(1, 256), W2, b2.reshape(1, H), degp, *g1)


def _tc_mid(p, xw, degp, gb, g2):
    return pl.pallas_call(
        _tc_mid_body,
        grid=(GRID,),
        in_specs=[
            pl.BlockSpec((NC, R, H), lambda i: (0, i, 0)),
            pl.BlockSpec((R, H), lambda i: (i, 0)),
            pl.BlockSpec((NC, R), lambda i: (0, i)),
            _full((1, H)),
            _full((3, H, H)), _full((3, H)), _full((3, H)), _full((1, H)),
            _full((H, H, H)), _full((H, H)),
        ],
        out_specs=pl.BlockSpec((R, H), lambda i: (i, 0)),
        out_shape=jax.ShapeDtypeStruct((NP, H), jnp.float32),
    )(p, xw, degp, gb.reshape(1, H), *g2)


def _tc_out(q, xw, degp, gb, wp_vec, bp_sum):
    return pl.pallas_call(
        _tc_out_body,
        grid=(GRID,),
        in_specs=[
            pl.BlockSpec((NC, R, H), lambda i: (0, i, 0)),
            pl.BlockSpec((R, H), lambda i: (i, 0)),
            pl.BlockSpec((NC, R), lambda i: (0, i)),
            _full((1, H)), _full((1, H)), _full((1, 1)),
        ],
        out_specs=pl.BlockSpec((R,), lambda i: (i,)),
        out_shape=jax.ShapeDtypeStruct((NP,), jnp.float32),
    )(q, xw, degp, gb.reshape(1, H), wp_vec, bp_sum)


def _sc_mesh():
    return plsc.VectorSubcoreMesh(core_axis_name="c", subcore_axis_name="s",
                                  num_cores=NC, num_subcores=NS)


@functools.cache
def _build_sc_degree():
    @functools.partial(
        pl.kernel,
        out_type=jax.ShapeDtypeStruct((NC * NP,), jnp.float32),
        mesh=_sc_mesh(),
        scratch_types=[
            pltpu.VMEM_SHARED((NP,), jnp.float32),
            pltpu.VMEM((CHR, 128), jnp.int32),
            pltpu.VMEM((128,), jnp.float32),
            pltpu.SemaphoreType.DMA,
        ],
    )
    def sc_degree(colp2, zrow, ones128, degp, shared_deg, colv, onesv, sem):
        c = lax.axis_index("c")
        s = lax.axis_index("s")
        wid = s * NC + c
        pltpu.sync_copy(ones128, onesv)
        pltpu.sync_copy(zrow, shared_deg.at[pl.ds(s * NSL, NSL)])
        plsc.subcore_barrier()

        def body(t, carry):
            base = wid * RPW + t * CHR
            pltpu.sync_copy(colp2.at[pl.ds(base, CHR)], colv)
            descs = [
                pltpu.async_copy(onesv, shared_deg.at[colv.at[j]], sem,
                                 add=True)
                for j in range(CHR)
            ]
            for d in descs:
                d.wait()
            return carry

        lax.fori_loop(0, TIT, body, 0)
        plsc.subcore_barrier()
        pltpu.sync_copy(shared_deg.at[pl.ds(s * NSL, NSL)],
                        degp.at[pl.ds(c * NP + s * NSL, NSL)])

    return sc_degree


@functools.cache
def _build_sc_scatter():
    @functools.partial(
        pl.kernel,
        out_type=jax.ShapeDtypeStruct((NC * NP, H), jnp.float32),
        mesh=_sc_mesh(),
        compiler_params=pltpu.CompilerParams(use_tc_tiling_on_sc=False),
        scratch_types=[
            pltpu.VMEM_SHARED((NP, H), jnp.float32),
            pltpu.VMEM((CHR, 128), jnp.int32),
            pltpu.VMEM((CHR, 128), jnp.int32),
            pltpu.VMEM((CHR, 128, H), jnp.float32),
            pltpu.SemaphoreType.DMA,
            pltpu.SemaphoreType.DMA,
        ],
    )
    def sc_scatter(rowp2, colp2, table, zblk, pout, shared_agg, rowv, colv,
                   gat, semg, sems):
        c = lax.axis_index("c")
        s = lax.axis_index("s")
        wid = s * NC + c
        pltpu.sync_copy(zblk, shared_agg.at[pl.ds(s * NSL, NSL)])
        plsc.subcore_barrier()

        def body(t, carry):
            base = wid * RPW + t * CHR
            pltpu.sync_copy(rowp2.at[pl.ds(base, CHR)], rowv)
            pltpu.sync_copy(colp2.at[pl.ds(base, CHR)], colv)
            gd = [
                pltpu.async_copy(table.at[rowv.at[j]], gat.at[j], semg)
                for j in range(CHR)
            ]
            for d in gd:
                d.wait()
            sd = [
                pltpu.async_copy(gat.at[j], shared_agg.at[colv.at[j]], sems,
                                 add=True)
                for j in range(CHR)
            ]
            for d in sd:
                d.wait()
            return carry

        lax.fori_loop(0, TIT, body, 0)
        plsc.subcore_barrier()
        pltpu.sync_copy(shared_agg.at[pl.ds(s * NSL, NSL)],
                        pout.at[pl.ds(c * NP + s * NSL, NSL)])

    return sc_scatter


def kernel(x, edge_index, W1, b1, W2, b2, gru1_wih, gru1_whh, gru1_bih,
           gru1_bhh, wt1_w, wt1_b, gcn1_b, mem1, gru2_wih, gru2_whh, gru2_bih,
           gru2_bhh, wt2_w, wt2_b, gcn2_b, mem2, Wp, bp):
    row = edge_index[0]
    col = edge_index[1]
    pad = EP - E
    ar = jnp.arange(pad, dtype=jnp.int32)
    rowp2 = jnp.concatenate([row, ar % N]).reshape(EPR, 128)
    colp2 = jnp.concatenate([col, N + ar % (NP - N)]).reshape(EPR, 128)

    zrow = jnp.zeros((NSL,), jnp.float32)
    zblk = jnp.zeros((NSL, H), jnp.float32)
    ones128 = jnp.ones((128,), jnp.float32)

    g1 = (gru1_wih.reshape(3, H, H), gru1_bih.reshape(3, H),
          gru1_bhh.reshape(3, H), mem1.reshape(1, H),
          wt1_w.reshape(H, H, H), wt1_b.reshape(H, H))
    g2 = (gru2_wih.reshape(3, H, H), gru2_bih.reshape(3, H),
          gru2_bhh.reshape(3, H), mem2.reshape(1, H),
          wt2_w.reshape(H, H, H), wt2_b.reshape(H, H))

    degp = _build_sc_degree()(colp2, zrow, ones128).reshape(NC, NP)

    xw1 = _tc_a(x, W1, b1, W2, b2, degp, g1)
    sc_scatter = _build_sc_scatter()
    p = sc_scatter(rowp2, colp2, xw1, zblk).reshape(NC, NP, H)
    xw2 = _tc_mid(p, xw1, degp, gcn1_b, g2)
    q = sc_scatter(rowp2, colp2, xw2, zblk).reshape(NC, NP, H)

    wp_vec = (Wp[0] + Wp[1]).reshape(1, H)
    bp_sum = (bp[0] + bp[1]).reshape(1, 1)
    out1d = _tc_out(q, xw2, degp, gcn2_b, wp_vec, bp_sum)
    return out1d[:N]


# speedup vs baseline: 71.9977x; 1.1560x over previous
import functools

import jax
import jax.numpy as jnp
from jax import lax
from jax.experimental import pallas as pl
from jax.experimental.pallas import tpu as pltpu
from jax.experimental.pallas import tpu_sc as plsc

N = 100000
E = 3200000
D_IN = 128
H = 16

NC = 2
NS = 16
NW = NC * NS

R = 2048
GRID = 49
NP = R * GRID
NSL = NP // NS

EP = 3276800
EPR = EP // 128
RPW = EPR // NW
CHR = 8
DTIT = RPW // CHR // 2
SCH = 4
CPW = RPW // SCH
TIT = CPW // 2


def _leaky(v):
    return jnp.where(v >= 0, v, 0.01 * v)


def _dyn_weight(wih3, bih2, bhh2, mem2, wtw3, wtb2):
    m3 = mem2.reshape(1, 1, H)
    gi = jnp.sum(wih3 * m3, axis=-1) + bih2
    r = jax.nn.sigmoid(gi[0:1] + bhh2[0:1])
    z = jax.nn.sigmoid(gi[1:2] + bhh2[1:2])
    n = jnp.tanh(gi[2:3] + r * bhh2[2:3])
    upd = (1.0 - z) * n
    return jnp.sum(wtw3 * upd.reshape(1, 1, H), axis=-1) + wtb2


def _dis_from_deg(degr):
    deg = degr[0, :] + degr[1, :] + 1.0
    return lax.rsqrt(deg)


def _tc_mlp_body(xr, w1r, b1r, w2r, b2r, outr):
    h = _leaky(lax.dot_general(xr[...], w1r[...], (((1,), (1,)), ((), ())),
                               preferred_element_type=jnp.float32) + b1r[...])
    outr[...] = _leaky(lax.dot_general(h, w2r[...], (((1,), (1,)), ((), ())),
                                       preferred_element_type=jnp.float32)
                       + b2r[...])


def _tc_scale_body(hr, degr, wih3, bih2, bhh2, mem2, wtw3, wtb2, outr):
    dis = _dis_from_deg(degr[...])
    nw = _dyn_weight(wih3[...], bih2[...], bhh2[...], mem2[...], wtw3[...],
                     wtb2[...])
    xw = lax.dot_general(hr[...], nw, (((1,), (1,)), ((), ())),
                         preferred_element_type=jnp.float32)
    outr[...] = dis[:, None] * xw


def _tc_mid_body(pr, xwr, degr, gbr, wih3, bih2, bhh2, mem2, wtw3, wtb2, outr):
    dis = _dis_from_deg(degr[...])
    agg = dis[:, None] * (pr[0] + pr[1] + xwr[...]) + gbr[...]
    h = _leaky(agg)
    nw = _dyn_weight(wih3[...], bih2[...], bhh2[...], mem2[...], wtw3[...],
                     wtb2[...])
    xw = lax.dot_general(h, nw, (((1,), (1,)), ((), ())),
                         preferred_element_type=jnp.float32)
    outr[...] = dis[:, None] * xw


def _tc_out_body(qr, xwr, degr, gbr, wpr, bpr, outr):
    dis = _dis_from_deg(degr[...])
    h = _leaky(dis[:, None] * (qr[0] + qr[1] + xwr[...]) + gbr[...])
    outr[...] = jnp.sum(h * wpr[...], axis=1) + bpr[0, 0]


def _full(shape):
    return pl.BlockSpec(shape, lambda i: tuple(0 for _ in shape))


def _tc_mlp(x, W1, b1, W2, b2):
    return pl.pallas_call(
        _tc_mlp_body,
        grid=(GRID,),
        in_specs=[
            pl.BlockSpec((R, D_IN), lambda i: (i, 0)),
            _full((256, D_IN)), _full((1, 256)),
            _full((H, 256)), _full((1, H)),
        ],
        out_specs=pl.BlockSpec((R, H), lambda i: (i, 0)),
        out_shape=jax.ShapeDtypeStruct((NP, H), jnp.float32),
    )(x, W1, b1.reshape(1, 256), W2, b2.reshape(1, H))


def _tc_scale(h0, degp, g1):
    return pl.pallas_call(
        _tc_scale_body,
        grid=(GRID,),
        in_specs=[
            pl.BlockSpec((R, H), lambda i: (i, 0)),
            pl.BlockSpec((NC, R), lambda i: (0, i)),
            _full((3, H, H)), _full((3, H)), _full((3, H)), _full((1, H)),
            _full((H, H, H)), _full((H, H)),
        ],
        out_specs=pl.BlockSpec((R, H), lambda i: (i, 0)),
        out_shape=jax.ShapeDtypeStruct((NP, H), jnp.float32),
    )(h0, degp, *g1)


def _tc_mid(p, xw, degp, gb, g2):
    return pl.pallas_call(
        _tc_mid_body,
        grid=(GRID,),
        in_specs=[
            pl.BlockSpec((NC, R, H), lambda i: (0, i, 0)),
            pl.BlockSpec((R, H), lambda i: (i, 0)),
            pl.BlockSpec((NC, R), lambda i: (0, i)),
            _full((1, H)),
            _full((3, H, H)), _full((3, H)), _full((3, H)), _full((1, H)),
            _full((H, H, H)), _full((H, H)),
        ],
        out_specs=pl.BlockSpec((R, H), lambda i: (i, 0)),
        out_shape=jax.ShapeDtypeStruct((NP, H), jnp.float32),
    )(p, xw, degp, gb.reshape(1, H), *g2)


def _tc_out(q, xw, degp, gb, wp_vec, bp_sum):
    return pl.pallas_call(
        _tc_out_body,
        grid=(GRID,),
        in_specs=[
            pl.BlockSpec((NC, R, H), lambda i: (0, i, 0)),
            pl.BlockSpec((R, H), lambda i: (i, 0)),
            pl.BlockSpec((NC, R), lambda i: (0, i)),
            _full((1, H)), _full((1, H)), _full((1, 1)),
        ],
        out_specs=pl.BlockSpec((R,), lambda i: (i,)),
        out_shape=jax.ShapeDtypeStruct((NP,), jnp.float32),
    )(q, xw, degp, gb.reshape(1, H), wp_vec, bp_sum)


def _sc_mesh():
    return plsc.VectorSubcoreMesh(core_axis_name="c", subcore_axis_name="s",
                                  num_cores=NC, num_subcores=NS)


@functools.cache
def _build_sc_degree():
    @functools.partial(
        pl.kernel,
        out_type=jax.ShapeDtypeStruct((NC * NP,), jnp.float32),
        mesh=_sc_mesh(),
        scratch_types=[
            pltpu.VMEM_SHARED((NP,), jnp.float32),
            pltpu.VMEM((CHR, 128), jnp.int32),
            pltpu.VMEM((CHR, 128), jnp.int32),
            pltpu.VMEM((128,), jnp.float32),
            pltpu.SemaphoreType.DMA,
            pltpu.SemaphoreType.DMA,
        ],
    )
    def sc_degree(colp2, zrow, ones128, degp, shared_deg, colva, colvb,
                  onesv, sema, semb):
        c = lax.axis_index("c")
        s = lax.axis_index("s")
        wid = s * NC + c
        pltpu.sync_copy(ones128, onesv)
        pltpu.sync_copy(zrow, shared_deg.at[pl.ds(s * NSL, NSL)])
        plsc.subcore_barrier()

        def body(t, carry):
            base = wid * RPW + t * 2 * CHR
            pltpu.sync_copy(colp2.at[pl.ds(base, CHR)], colva)
            da = [
                pltpu.async_copy(onesv, shared_deg.at[colva.at[j]], sema,
                                 add=True)
                for j in range(CHR)
            ]
            pltpu.sync_copy(colp2.at[pl.ds(base + CHR, CHR)], colvb)
            db = [
                pltpu.async_copy(onesv, shared_deg.at[colvb.at[j]], semb,
                                 add=True)
                for j in range(CHR)
            ]
            for d in da:
                d.wait()
            for d in db:
                d.wait()
            return carry

        lax.fori_loop(0, DTIT, body, 0)
        plsc.subcore_barrier()
        pltpu.sync_copy(shared_deg.at[pl.ds(s * NSL, NSL)],
                        degp.at[pl.ds(c * NP + s * NSL, NSL)])

    return sc_degree


@functools.cache
def _build_sc_scatter():
    @functools.partial(
        pl.kernel,
        out_type=jax.ShapeDtypeStruct((NC * NP, H), jnp.float32),
        mesh=_sc_mesh(),
        compiler_params=pltpu.CompilerParams(use_tc_tiling_on_sc=False),
        scratch_types=[
            pltpu.VMEM_SHARED((NP, H), jnp.float32),
            pltpu.VMEM((2 * SCH, 128), jnp.int32),
            pltpu.VMEM((2 * SCH, 128), jnp.int32),
            pltpu.VMEM((SCH, 128, H), jnp.float32),
            pltpu.VMEM((SCH, 128, H), jnp.float32),
            pltpu.SemaphoreType.DMA,
            pltpu.SemaphoreType.DMA,
            pltpu.SemaphoreType.DMA,
            pltpu.SemaphoreType.DMA,
        ],
    )
    def sc_scatter(pkd, table, zblk, pout, shared_agg, rcva, rcvb, gata, gatb,
                   semga, semgb, semsa, semsb):
        c = lax.axis_index("c")
        s = lax.axis_index("s")
        wid = s * NC + c
        pltpu.sync_copy(zblk, shared_agg.at[pl.ds(s * NSL, NSL)])
        plsc.subcore_barrier()

        def body(t, carry):
            base = (wid * CPW + 2 * t) * 2 * SCH
            pltpu.sync_copy(pkd.at[pl.ds(base, 2 * SCH)], rcva)
            ga = [
                pltpu.async_copy(table.at[rcva.at[j]], gata.at[j], semga)
                for j in range(SCH)
            ]
            pltpu.sync_copy(pkd.at[pl.ds(base + 2 * SCH, 2 * SCH)], rcvb)
            gb = [
                pltpu.async_copy(table.at[rcvb.at[j]], gatb.at[j], semgb)
                for j in range(SCH)
            ]
            for d in ga:
                d.wait()
            sa = [
                pltpu.async_copy(gata.at[j], shared_agg.at[rcva.at[SCH + j]],
                                 semsa, add=True)
                for j in range(SCH)
            ]
            for d in gb:
                d.wait()
            sb = [
                pltpu.async_copy(gatb.at[j], shared_agg.at[rcvb.at[SCH + j]],
                                 semsb, add=True)
                for j in range(SCH)
            ]
            for d in sa:
                d.wait()
            for d in sb:
                d.wait()
            return carry

        lax.fori_loop(0, TIT, body, 0)
        plsc.subcore_barrier()
        pltpu.sync_copy(shared_agg.at[pl.ds(s * NSL, NSL)],
                        pout.at[pl.ds(c * NP + s * NSL, NSL)])

    return sc_scatter


def kernel(x, edge_index, W1, b1, W2, b2, gru1_wih, gru1_whh, gru1_bih,
           gru1_bhh, wt1_w, wt1_b, gcn1_b, mem1, gru2_wih, gru2_whh, gru2_bih,
           gru2_bhh, wt2_w, wt2_b, gcn2_b, mem2, Wp, bp):
    row = edge_index[0]
    col = edge_index[1]
    pad = EP - E
    ar = jnp.arange(pad, dtype=jnp.int32)
    rowp2 = jnp.concatenate([row, ar % N]).reshape(EPR, 128)
    colp2 = jnp.concatenate([col, N + ar % (NP - N)]).reshape(EPR, 128)
    pkd = jnp.concatenate(
        [rowp2.reshape(-1, SCH, 128), colp2.reshape(-1, SCH, 128)],
        axis=1).reshape(-1, 128)

    zrow = jnp.zeros((NSL,), jnp.float32)
    zblk = jnp.zeros((NSL, H), jnp.float32)
    ones128 = jnp.ones((128,), jnp.float32)

    g1 = (gru1_wih.reshape(3, H, H), gru1_bih.reshape(3, H),
          gru1_bhh.reshape(3, H), mem1.reshape(1, H),
          wt1_w.reshape(H, H, H), wt1_b.reshape(H, H))
    g2 = (gru2_wih.reshape(3, H, H), gru2_bih.reshape(3, H),
          gru2_bhh.reshape(3, H), mem2.reshape(1, H),
          wt2_w.reshape(H, H, H), wt2_b.reshape(H, H))

    degp = _build_sc_degree()(colp2, zrow, ones128).reshape(NC, NP)
    h0 = _tc_mlp(x, W1, b1, W2, b2)

    xw1 = _tc_scale(h0, degp, g1)
    sc_scatter = _build_sc_scatter()
    p = sc_scatter(pkd, xw1, zblk).reshape(NC, NP, H)
    xw2 = _tc_mid(p, xw1, degp, gcn1_b, g2)
    q = sc_scatter(pkd, xw2, zblk).reshape(NC, NP, H)

    wp_vec = (Wp[0] + Wp[1]).reshape(1, H)
    bp_sum = (bp[0] + bp[1]).reshape(1, 1)
    out1d = _tc_out(q, xw2, degp, gcn2_b, wp_vec, bp_sum)
    return out1d[:N]
